# Initial kernel scaffold; baseline (speedup 1.0000x reference)
import jax
import jax.numpy as jnp
from jax.experimental import pallas as pl


def kernel(atom_node_attribute, atom_edge, atom_edge_attribute, atom_batch, frag_node_attribute, frag_edge, node_batch, atom_We, atom_W1, atom_b1, atom_W2, atom_b2, frag_W1, frag_b1, frag_W2, frag_b2):
    raise NotImplementedError("write your pallas kernel here")


# speedup vs baseline: 2.0680x; 2.0680x over previous
import functools

import jax
import jax.numpy as jnp
from jax import lax
from jax.experimental import pallas as pl
from jax.experimental.pallas import tpu as pltpu
from jax.experimental.pallas import tpu_sc as plsc

NC = 2
NS = 16
NW = NC * NS
LANES = 16

F32 = jnp.float32


def _sc_edge_scatter(n_nodes, n_edges, d, chunk, has_edge_term):
    ept = n_edges // NW
    n_chunks = ept // chunk
    assert ept % chunk == 0 and chunk % 8 == 0 and chunk <= 128
    rpt = (n_nodes // NS) // 8 * 8
    extra = n_nodes - NS * rpt
    vecs = d // LANES

    mesh = plsc.VectorSubcoreMesh(
        core_axis_name="c", subcore_axis_name="s",
        num_cores=NC, num_subcores=NS)

    def body(*refs):
        if has_edge_term:
            (x_hbm, e_hbm, src_hbm, dst_hbm, out_hbm,
             acc, src_v, dst_v, xg, ev, msg, sem) = refs
        else:
            (x_hbm, src_hbm, dst_hbm, out_hbm,
             acc, src_v, dst_v, xg, msg, sem) = refs
            ev = None
        cid = lax.axis_index("c")
        sid = lax.axis_index("s")
        wid = cid * NS + sid

        def zero_row(i, _):
            for j in range(vecs):
                msg[i, pl.ds(j * LANES, LANES)] = jnp.zeros((LANES,), F32)
            return 0
        lax.fori_loop(0, chunk, zero_row, 0)
        zbase = sid * rpt
        full, rem = divmod(rpt, chunk)
        for t in range(full):
            pltpu.sync_copy(msg.at[pl.ds(0, chunk)],
                            acc.at[pl.ds(zbase + t * chunk, chunk)])
        if rem:
            pltpu.sync_copy(msg.at[pl.ds(0, rem)],
                            acc.at[pl.ds(zbase + full * chunk, rem)])
        if extra:
            @pl.when(sid == NS - 1)
            def _():
                pltpu.sync_copy(msg.at[pl.ds(0, extra)],
                                acc.at[pl.ds(NS * rpt, extra)])
        plsc.subcore_barrier()

        def chunk_body(g, _):
            ebase = wid * ept + g * chunk
            pltpu.sync_copy(src_hbm.at[pl.ds(ebase, chunk)], src_v)
            pltpu.sync_copy(dst_hbm.at[pl.ds(ebase, chunk)], dst_v)
            pltpu.async_copy(x_hbm.at[src_v], xg, sem).wait()
            if ev is not None:
                pltpu.sync_copy(e_hbm.at[pl.ds(ebase, chunk)], ev)

            def row_body(i, _):
                for j in range(vecs):
                    s = pl.ds(j * LANES, LANES)
                    m = xg[i, s] if ev is None else xg[i, s] + ev[i, s]
                    msg[i, s] = jnp.maximum(m, 0.0)
                return 0
            lax.fori_loop(0, chunk, row_body, 0)
            pltpu.sync_copy(msg, acc.at[dst_v], add=True)
            return 0
        lax.fori_loop(0, n_chunks, chunk_body, 0)

        plsc.subcore_barrier()
        pltpu.sync_copy(acc.at[pl.ds(sid * rpt, rpt)],
                        out_hbm.at[cid, pl.ds(sid * rpt, rpt)])
        if extra:
            @pl.when(sid == NS - 1)
            def _():
                pltpu.sync_copy(acc.at[pl.ds(NS * rpt, extra)],
                                out_hbm.at[cid, pl.ds(NS * rpt, extra)])

    scratch = [
        pltpu.VMEM_SHARED((n_nodes, d), F32),
        pltpu.VMEM((chunk,), jnp.int32),
        pltpu.VMEM((chunk,), jnp.int32),
        pltpu.VMEM((chunk, d), F32),
    ]
    if has_edge_term:
        scratch.append(pltpu.VMEM((chunk, d), F32))
    scratch += [
        pltpu.VMEM((chunk, d), F32),
        pltpu.SemaphoreType.DMA,
    ]

    return pl.kernel(
        body,
        out_type=jax.ShapeDtypeStruct((NC, n_nodes, d), F32),
        mesh=mesh,
        scratch_types=scratch,
    )


def _edge_encode(edge_attr, We):
    la, de, d = We.shape
    e_total = edge_attr.shape[0]
    blk = 4000
    nblk = e_total // blk

    def body(ea_ref, we_ref, out_ref):
        out_ref[0] = jnp.dot(ea_ref[...], we_ref[0],
                             preferred_element_type=F32)

    return pl.pallas_call(
        body,
        grid=(la, nblk),
        in_specs=[
            pl.BlockSpec((blk, de), lambda l, i: (i, 0)),
            pl.BlockSpec((1, de, d), lambda l, i: (l, 0, 0)),
        ],
        out_specs=pl.BlockSpec((1, blk, d), lambda l, i: (l, i, 0)),
        out_shape=jax.ShapeDtypeStruct((la, e_total, d), F32),
    )(edge_attr, We)


def _mlp(x, pA, pB, W1, b1, W2, b2, relu_out, blk):
    n, d = x.shape
    nblk = n // blk

    def body(x_ref, pa_ref, pb_ref, w1_ref, b1_ref, w2_ref, b2_ref, out_ref):
        t = x_ref[...] + pa_ref[...] + pb_ref[...]
        h = jnp.maximum(jnp.dot(t, w1_ref[...], preferred_element_type=F32)
                        + b1_ref[...], 0.0)
        y = jnp.dot(h, w2_ref[...], preferred_element_type=F32) + b2_ref[...]
        if relu_out:
            y = jnp.maximum(y, 0.0)
        out_ref[...] = y

    row = lambda i: (i, 0)
    fixed = lambda i: (0, 0)
    return pl.pallas_call(
        body,
        grid=(nblk,),
        in_specs=[
            pl.BlockSpec((blk, d), row),
            pl.BlockSpec((blk, d), row),
            pl.BlockSpec((blk, d), row),
            pl.BlockSpec(W1.shape, fixed),
            pl.BlockSpec(b1.shape, fixed),
            pl.BlockSpec(W2.shape, fixed),
            pl.BlockSpec(b2.shape, fixed),
        ],
        out_specs=pl.BlockSpec((blk, d), row),
        out_shape=jax.ShapeDtypeStruct((n, d), F32),
    )(x, pA, pB, W1, b1, W2, b2)


def _mlp_pool(x, pA, pB, W1, b1, W2, b2, seg2d, nseg, blk):
    n, d = x.shape
    nblk = n // blk

    def body(x_ref, pa_ref, pb_ref, w1_ref, b1_ref, w2_ref, b2_ref,
             seg_ref, out_ref):
        i = pl.program_id(0)
        t = x_ref[...] + pa_ref[...] + pb_ref[...]
        h = jnp.maximum(jnp.dot(t, w1_ref[...], preferred_element_type=F32)
                        + b1_ref[...], 0.0)
        y = jnp.dot(h, w2_ref[...], preferred_element_type=F32) + b2_ref[...]
        seg = seg_ref[0, 0, :]
        rows = lax.broadcasted_iota(jnp.int32, (nseg, blk), 0)
        P = (rows == seg[None, :]).astype(F32)
        aug = jnp.concatenate([y, jnp.ones((blk, d), F32)], axis=1)
        contrib = jnp.dot(P, aug, preferred_element_type=F32)

        @pl.when(i == 0)
        def _():
            out_ref[...] = jnp.zeros_like(out_ref)
        out_ref[...] += contrib

    row = lambda i: (i, 0)
    fixed = lambda i: (0, 0)
    return pl.pallas_call(
        body,
        grid=(nblk,),
        in_specs=[
            pl.BlockSpec((blk, d), row),
            pl.BlockSpec((blk, d), row),
            pl.BlockSpec((blk, d), row),
            pl.BlockSpec(W1.shape, fixed),
            pl.BlockSpec(b1.shape, fixed),
            pl.BlockSpec(W2.shape, fixed),
            pl.BlockSpec(b2.shape, fixed),
            pl.BlockSpec((1, 1, blk), lambda i: (i, 0, 0)),
        ],
        out_specs=pl.BlockSpec((nseg, 2 * d), fixed),
        out_shape=jax.ShapeDtypeStruct((nseg, 2 * d), F32),
    )(x, pA, pB, W1, b1, W2, b2, seg2d)


def _finalize(pa, pf, d):
    nseg = pa.shape[0]

    def body(pa_ref, pf_ref, out_ref):
        out_ref[:, :d] = pa_ref[:, :d] / jnp.maximum(pa_ref[:, d:], 1.0)
        out_ref[:, d:] = pf_ref[:, :d] / jnp.maximum(pf_ref[:, d:], 1.0)

    return pl.pallas_call(
        body,
        out_shape=jax.ShapeDtypeStruct((nseg, 2 * d), F32),
    )(pa, pf)


def kernel(atom_node_attribute, atom_edge, atom_edge_attribute, atom_batch,
           frag_node_attribute, frag_edge, node_batch,
           atom_We, atom_W1, atom_b1, atom_W2, atom_b2,
           frag_W1, frag_b1, frag_W2, frag_b2):
    n_atom, d = atom_node_attribute.shape
    e_atom = atom_edge.shape[1]
    n_frag = frag_node_attribute.shape[0]
    e_frag = frag_edge.shape[1]
    la = atom_We.shape[0]
    lf = frag_W1.shape[0]
    nseg = 512

    e_all = _edge_encode(atom_edge_attribute, atom_We)

    a_src, a_dst = atom_edge[0], atom_edge[1]
    f_src, f_dst = frag_edge[0], frag_edge[1]

    atom_scatter = _sc_edge_scatter(n_atom, e_atom, d, chunk=80,
                                    has_edge_term=True)
    frag_scatter = _sc_edge_scatter(n_frag, e_frag, d, chunk=128,
                                    has_edge_term=False)

    atom_blk = 1000
    frag_blk = 1024
    atom_seg2d = atom_batch.reshape(n_atom // atom_blk, 1, atom_blk)
    frag_seg2d = node_batch.reshape(n_frag // frag_blk, 1, frag_blk)

    x = atom_node_attribute
    for l in range(la):
        partials = atom_scatter(x, e_all[l], a_src, a_dst)
        w1, b1 = atom_W1[l], atom_b1[l].reshape(1, -1)
        w2, b2 = atom_W2[l], atom_b2[l].reshape(1, -1)
        if l < la - 1:
            x = _mlp(x, partials[0], partials[1], w1, b1, w2, b2,
                     relu_out=True, blk=atom_blk)
        else:
            pa = _mlp_pool(x, partials[0], partials[1], w1, b1, w2, b2,
                           atom_seg2d, nseg, blk=atom_blk)

    y = frag_node_attribute
    for l in range(lf):
        partials = frag_scatter(y, f_src, f_dst)
        w1, b1 = frag_W1[l], frag_b1[l].reshape(1, -1)
        w2, b2 = frag_W2[l], frag_b2[l].reshape(1, -1)
        if l < lf - 1:
            y = _mlp(y, partials[0], partials[1], w1, b1, w2, b2,
                     relu_out=True, blk=frag_blk)
        else:
            pf = _mlp_pool(y, partials[0], partials[1], w1, b1, w2, b2,
                           frag_seg2d, nseg, blk=frag_blk)

    return _finalize(pa, pf, d)
